# baseline (speedup 1.0000x reference)
import functools

import jax
import jax.numpy as jnp
from jax import lax
from jax.experimental import pallas as pl
from jax.experimental.pallas import tpu as pltpu
from jax.experimental.pallas import tpu_sc as plsc

_NC = 2
_NS = 16
_NW = _NC * _NS

_CH = 32
_NSLOT = 3


def _sc_copy_body(s, e, tab_hbm, out_hbm, buf, in_sems, out_sems):
    rows_per_w = s // _NW
    nchunk = rows_per_w // _CH
    wid = lax.axis_index("s") * _NC + lax.axis_index("c")
    base = wid * rows_per_w

    def start_in(c, slot):
        cp = pltpu.make_async_copy(
            tab_hbm.at[pl.ds(base + c * _CH, _CH)], buf.at[slot],
            in_sems.at[slot])
        cp.start()
        return cp

    def start_out(c, slot):
        cp = pltpu.make_async_copy(
            buf.at[slot], out_hbm.at[pl.ds(base + c * _CH, _CH)],
            out_sems.at[slot])
        cp.start()
        return cp

    ins = [None] * nchunk
    outs = [None] * nchunk
    for c in range(min(_NSLOT, nchunk)):
        ins[c] = start_in(c, c)
    for c in range(nchunk):
        slot = c % _NSLOT
        ins[c].wait()
        outs[c] = start_out(c, slot)
        nxt = c + _NSLOT
        if nxt < nchunk:
            outs[c].wait()
            ins[nxt] = start_in(nxt, slot)
    for c in range(max(nchunk - _NSLOT, 0), nchunk):
        outs[c].wait()


def kernel(x, pos_table):
    s = x.shape[0]
    n, e = pos_table.shape
    mesh = plsc.VectorSubcoreMesh(core_axis_name="c", subcore_axis_name="s")
    k = pl.kernel(
        functools.partial(_sc_copy_body, s, e),
        out_type=jax.ShapeDtypeStruct((s, e), pos_table.dtype),
        mesh=mesh,
        scratch_types=[
            pltpu.VMEM((_NSLOT, _CH, e), pos_table.dtype),
            pltpu.SemaphoreType.DMA((_NSLOT,)),
            pltpu.SemaphoreType.DMA((_NSLOT,)),
        ],
    )
    out = k(pos_table)
    return out.reshape(s, 1, e)


# speedup vs baseline: 1.5704x; 1.5704x over previous
import functools

import jax
import jax.numpy as jnp
from jax import lax
from jax.experimental import pallas as pl
from jax.experimental.pallas import tpu as pltpu
from jax.experimental.pallas import tpu_sc as plsc

_NC = 2
_NS = 16
_NW = _NC * _NS

_CH = 32
_NSLOT = 3


def _sc_copy_body(s, e, tab_hbm, out_hbm, buf, in_sems, out_sems):
    rows_per_w = s // _NW
    nchunk = rows_per_w // _CH
    wid = lax.axis_index("s") * _NC + lax.axis_index("c")
    base = wid * rows_per_w

    def start_in(c, slot):
        cp = pltpu.make_async_copy(
            tab_hbm.at[pl.ds(base + c * _CH, _CH)], buf.at[slot, :, 0],
            in_sems.at[slot])
        cp.start()
        return cp

    def start_out(c, slot):
        cp = pltpu.make_async_copy(
            buf.at[slot], out_hbm.at[pl.ds(base + c * _CH, _CH)],
            out_sems.at[slot])
        cp.start()
        return cp

    ins = [None] * nchunk
    outs = [None] * nchunk
    for c in range(min(_NSLOT, nchunk)):
        ins[c] = start_in(c, c)
    for c in range(nchunk):
        slot = c % _NSLOT
        ins[c].wait()
        outs[c] = start_out(c, slot)
        nxt = c + _NSLOT
        if nxt < nchunk:
            outs[c].wait()
            ins[nxt] = start_in(nxt, slot)
    for c in range(max(nchunk - _NSLOT, 0), nchunk):
        outs[c].wait()


def kernel(x, pos_table):
    s = x.shape[0]
    n, e = pos_table.shape
    mesh = plsc.VectorSubcoreMesh(core_axis_name="c", subcore_axis_name="s")
    k = pl.kernel(
        functools.partial(_sc_copy_body, s, e),
        out_type=jax.ShapeDtypeStruct((s, 1, e), pos_table.dtype),
        mesh=mesh,
        scratch_types=[
            pltpu.VMEM((_NSLOT, _CH, 1, e), pos_table.dtype),
            pltpu.SemaphoreType.DMA((_NSLOT,)),
            pltpu.SemaphoreType.DMA((_NSLOT,)),
        ],
    )
    return k(pos_table)


# speedup vs baseline: 4.3162x; 2.7486x over previous
import jax
import jax.numpy as jnp
from jax.experimental import pallas as pl
from jax.experimental.pallas import tpu as pltpu

_NCH = 8


def _copy_body(tab_ref, out_ref, buf, in_sems, out_sems):
    s = out_ref.shape[0]
    ch = s // _NCH
    ins, outs = [], []
    for i in range(_NCH):
        c = pltpu.make_async_copy(
            tab_ref.at[pl.ds(i * ch, ch)], buf.at[i, :, 0], in_sems.at[i])
        c.start()
        ins.append(c)
    for i in range(_NCH):
        ins[i].wait()
        c = pltpu.make_async_copy(
            buf.at[i], out_ref.at[pl.ds(i * ch, ch)], out_sems.at[i])
        c.start()
        outs.append(c)
    for c in outs:
        c.wait()


def kernel(x, pos_table):
    s = x.shape[0]
    n, e = pos_table.shape
    out = pl.pallas_call(
        _copy_body,
        in_specs=[pl.BlockSpec(memory_space=pl.ANY)],
        out_specs=pl.BlockSpec(memory_space=pl.ANY),
        out_shape=jax.ShapeDtypeStruct((s, 1, e), pos_table.dtype),
        scratch_shapes=[
            pltpu.VMEM((_NCH, s // _NCH, 1, e), pos_table.dtype),
            pltpu.SemaphoreType.DMA((_NCH,)),
            pltpu.SemaphoreType.DMA((_NCH,)),
        ],
    )(pos_table)
    return out
